# Initial kernel scaffold; baseline (speedup 1.0000x reference)
import jax
import jax.numpy as jnp
from jax.experimental import pallas as pl


def kernel(x, edge_index, batch, enc_W, enc_b, res_W, res_b, conv_W, conv_b, dec_W, dec_b):
    raise NotImplementedError("write your pallas kernel here")


# speedup vs baseline: 7.5488x; 7.5488x over previous
import functools

import jax
import jax.numpy as jnp
from jax import lax
from jax.experimental import pallas as pl
from jax.experimental.pallas import tpu as pltpu
from jax.experimental.pallas import tpu_sc as plsc

N = 10000
NPAD = 10240
F = 128
G = 16
NLAYERS = 4
BLK = 1024
NBLK = NPAD // BLK
CHUNK = 128
NC = 2
NS = 16
NW = NC * NS
RPT = NPAD // NS

@functools.lru_cache(maxsize=1)
def _sc_kernels():
    mesh = plsc.VectorSubcoreMesh(core_axis_name="c", subcore_axis_name="s")

    @functools.partial(
        pl.kernel,
        mesh=mesh,
        out_type=jax.ShapeDtypeStruct((NC, NPAD, F), jnp.float32),
        scratch_types=[
            pltpu.VMEM((CHUNK,), jnp.int32),
            pltpu.VMEM((CHUNK, F), jnp.float32),
            pltpu.VMEM_SHARED((NPAD, F), jnp.float32),
        ],
    )
    def _deg_sc(dst_hbm, ones_hbm, zeros_hbm, cnt_hbm, idx_v, ones_v, acc):
        c = lax.axis_index("c")
        s = lax.axis_index("s")
        wid = c * NS + s
        epw = dst_hbm.shape[0] // NW
        pltpu.sync_copy(zeros_hbm, acc.at[pl.ds(s * RPT, RPT)])
        pltpu.sync_copy(ones_hbm, ones_v)
        plsc.subcore_barrier()

        def body(j, carry):
            base = wid * epw + j * CHUNK
            pltpu.sync_copy(dst_hbm.at[pl.ds(base, CHUNK)], idx_v)
            pltpu.sync_copy(ones_v, acc.at[idx_v], add=True)
            return carry

        lax.fori_loop(0, epw // CHUNK, body, 0)
        plsc.subcore_barrier()
        pltpu.sync_copy(acc.at[pl.ds(s * RPT, RPT)],
                        cnt_hbm.at[c, pl.ds(s * RPT, RPT)])

    @functools.partial(
        pl.kernel,
        mesh=mesh,
        out_type=jax.ShapeDtypeStruct((NC, NPAD, F), jnp.float32),
        scratch_types=[
            pltpu.VMEM((CHUNK,), jnp.int32),
            pltpu.VMEM((CHUNK,), jnp.int32),
            pltpu.VMEM((CHUNK, F), jnp.float32),
            pltpu.VMEM_SHARED((NPAD, F), jnp.float32),
            pltpu.SemaphoreType.DMA,
        ],
    )
    def _spmm_sc(u_hbm, src_hbm, dst_hbm, zeros_hbm, s_hbm,
                 src_v, dst_v, rows_v, acc, sem):
        c = lax.axis_index("c")
        s = lax.axis_index("s")
        wid = c * NS + s
        epw = src_hbm.shape[0] // NW
        pltpu.sync_copy(zeros_hbm, acc.at[pl.ds(s * RPT, RPT)])
        plsc.subcore_barrier()

        def body(j, carry):
            base = wid * epw + j * CHUNK
            pltpu.sync_copy(src_hbm.at[pl.ds(base, CHUNK)], src_v)
            pltpu.sync_copy(dst_hbm.at[pl.ds(base, CHUNK)], dst_v)
            pltpu.async_copy(u_hbm.at[src_v], rows_v, sem).wait()
            pltpu.sync_copy(rows_v, acc.at[dst_v], add=True)
            return carry

        lax.fori_loop(0, epw // CHUNK, body, 0)
        plsc.subcore_barrier()
        pltpu.sync_copy(acc.at[pl.ds(s * RPT, RPT)],
                        s_hbm.at[c, pl.ds(s * RPT, RPT)])

    return _deg_sc, _spmm_sc


def _dinv_block(cnt_ref, valid):
    cntv = cnt_ref[0, :, 0:1] + cnt_ref[1, :, 0:1]
    return jnp.where(valid, lax.rsqrt(cntv + 1.0), 0.0)


def _enc_body(x_ref, cnt_ref, encW_ref, encb_ref, convW_ref, resW_ref,
              cb_ref, rb_ref, u_ref, v_ref):
    i = pl.program_id(0)
    rows = i * BLK + lax.broadcasted_iota(jnp.int32, (BLK, 1), 0)
    valid = rows < N
    dinv = _dinv_block(cnt_ref, valid)
    x0 = jnp.dot(x_ref[...], encW_ref[...], preferred_element_type=jnp.float32)
    x0 = jnp.where(valid, x0 + encb_ref[...], 0.0)
    xw = jnp.dot(x0, convW_ref[...], preferred_element_type=jnp.float32)
    xr = jnp.dot(x0, resW_ref[...], preferred_element_type=jnp.float32)
    u_ref[...] = dinv * xw
    v_ref[...] = xr - xw + cb_ref[...] + rb_ref[...]


def _layer_body(s_ref, u_ref, v_ref, cnt_ref, convW_ref, resW_ref,
                cb_ref, rb_ref, uo_ref, vo_ref):
    i = pl.program_id(0)
    rows = i * BLK + lax.broadcasted_iota(jnp.int32, (BLK, 1), 0)
    valid = rows < N
    dinv = _dinv_block(cnt_ref, valid)
    z = dinv * (s_ref[0] + s_ref[1] + u_ref[...]) + v_ref[...]
    z = jnp.where(valid, jnp.maximum(z, 0.0), 0.0)
    xw = jnp.dot(z, convW_ref[...], preferred_element_type=jnp.float32)
    xr = jnp.dot(z, resW_ref[...], preferred_element_type=jnp.float32)
    uo_ref[...] = dinv * xw
    vo_ref[...] = xr - xw + cb_ref[...] + rb_ref[...]


def _final_body(s_ref, u_ref, v_ref, cnt_ref, b_ref, decW_ref, decb_ref,
                out_ref, pooled, cntg):
    i = pl.program_id(0)

    @pl.when(i == 0)
    def _init():
        pooled[...] = jnp.zeros((G, F), jnp.float32)
        cntg[...] = jnp.zeros((G, F), jnp.float32)

    rows = i * BLK + lax.broadcasted_iota(jnp.int32, (BLK, 1), 0)
    valid = rows < N
    dinv = _dinv_block(cnt_ref, valid)
    z = dinv * (s_ref[0] + s_ref[1] + u_ref[...]) + v_ref[...]
    z = jnp.where(valid, jnp.maximum(z, 0.0), 0.0)
    bb = b_ref[0]
    gid = lax.broadcasted_iota(jnp.int32, (G, BLK), 0)
    onehot = (bb == gid).astype(jnp.float32)
    pooled[...] = pooled[...] + jnp.dot(onehot, z, preferred_element_type=jnp.float32)
    cntg[...] = cntg[...] + jnp.sum(onehot, axis=1, keepdims=True)

    @pl.when(i == NBLK - 1)
    def _fin():
        out_ref[...] = (jnp.dot(pooled[...], decW_ref[...],
                                preferred_element_type=jnp.float32)
                        + cntg[...][:, 0:1] * decb_ref[...])


_w_spec = pl.BlockSpec((F, F), lambda i: (0, 0))
_b_spec = pl.BlockSpec((1, F), lambda i: (0, 0))
_row_spec = pl.BlockSpec((BLK, F), lambda i: (i, 0))
_cnt_spec = pl.BlockSpec((NC, BLK, F), lambda i: (0, i, 0))
_s_spec = pl.BlockSpec((NC, BLK, F), lambda i: (0, i, 0))
_uv_shape = [jax.ShapeDtypeStruct((NPAD, F), jnp.float32)] * 2

_enc_call = pl.pallas_call(
    _enc_body,
    grid=(NBLK,),
    in_specs=[_row_spec, _cnt_spec, _w_spec, _b_spec, _w_spec, _w_spec,
              _b_spec, _b_spec],
    out_specs=[_row_spec, _row_spec],
    out_shape=_uv_shape,
)

_layer_call = pl.pallas_call(
    _layer_body,
    grid=(NBLK,),
    in_specs=[_s_spec, _row_spec, _row_spec, _cnt_spec, _w_spec, _w_spec,
              _b_spec, _b_spec],
    out_specs=[_row_spec, _row_spec],
    out_shape=_uv_shape,
)

_final_call = pl.pallas_call(
    _final_body,
    grid=(NBLK,),
    in_specs=[_s_spec, _row_spec, _row_spec, _cnt_spec,
              pl.BlockSpec((1, 1, BLK), lambda i: (i, 0, 0)),
              _w_spec, _b_spec],
    out_specs=pl.BlockSpec((G, F), lambda i: (0, 0)),
    out_shape=jax.ShapeDtypeStruct((G, F), jnp.float32),
    scratch_shapes=[pltpu.VMEM((G, F), jnp.float32),
                    pltpu.VMEM((G, F), jnp.float32)],
)


def kernel(x, edge_index, batch, enc_W, enc_b, res_W, res_b, conv_W, conv_b,
           dec_W, dec_b):
    f32 = jnp.float32
    src = edge_index[0].astype(jnp.int32)
    dst = edge_index[1].astype(jnp.int32)
    e = src.shape[0]
    epw = -(-e // (NW * CHUNK)) * CHUNK
    epad = epw * NW
    pad = epad - e
    src_p = jnp.concatenate([src, jnp.full((pad,), N, jnp.int32)])
    dst_p = jnp.concatenate([dst, jnp.full((pad,), N, jnp.int32)])

    onesF = jnp.ones((CHUNK, F), f32)
    zerosF = jnp.zeros((RPT, F), f32)

    _deg_sc, _spmm_sc = _sc_kernels()
    cnt = _deg_sc(dst_p, onesF, zerosF)

    encb2 = enc_b.reshape(1, F).astype(f32)
    cb2 = conv_b.reshape(1, F).astype(f32)
    rb2 = res_b.reshape(1, F).astype(f32)

    U, V = _enc_call(x.astype(f32), cnt, enc_W.astype(f32), encb2,
                     conv_W.astype(f32), res_W.astype(f32), cb2, rb2)
    for _ in range(NLAYERS - 1):
        S = _spmm_sc(U, src_p, dst_p, zerosF)
        U, V = _layer_call(S, U, V, cnt, conv_W.astype(f32),
                           res_W.astype(f32), cb2, rb2)
    S = _spmm_sc(U, src_p, dst_p, zerosF)

    batch_p = jnp.concatenate(
        [batch.astype(jnp.int32), jnp.full((NPAD - N,), G, jnp.int32)]
    ).reshape(NBLK, 1, BLK)
    decW_p = jnp.pad(dec_W.astype(f32), ((0, 0), (0, F - dec_W.shape[1])))
    decb_p = jnp.pad(dec_b.astype(f32), (0, F - dec_b.shape[0])).reshape(1, F)

    out = _final_call(S, U, V, cnt, batch_p, decW_p, decb_p)
    return out[:, 0]
